# Initial kernel scaffold; baseline (speedup 1.0000x reference)
import jax
import jax.numpy as jnp
from jax.experimental import pallas as pl


def kernel(x, edge_index, W_in, W_out):
    raise NotImplementedError("write your pallas kernel here")


# speedup vs baseline: 4.2788x; 4.2788x over previous
import jax
import jax.numpy as jnp
from jax import lax
from jax.experimental import pallas as pl
from jax.experimental.pallas import tpu as pltpu
from jax.experimental.pallas import tpu_sc as plsc

N_NODES = 10000
N_EDGES = 320000
FEAT = 128

NC = 2
NS = 16
NW = NC * NS
CHUNK = 128

EDGES_PER_W = ((N_EDGES + NW * CHUNK - 1) // (NW * CHUNK)) * CHUNK
N_CHUNKS = EDGES_PER_W // CHUNK
E_PAD = EDGES_PER_W * NW
N_PAD = 10240
ROWS_PER_TILE = N_PAD // NS
CNT_W = 128

_MESH = plsc.VectorSubcoreMesh(core_axis_name="c", subcore_axis_name="s")


def _agg_body(src_h, dst_h, table_h, zeros_h,
              out_sum, src_v, dst_v, rows_v, acc_sh, sem):
    c = lax.axis_index("c")
    s = lax.axis_index("s")
    wid = c * NS + s

    pltpu.sync_copy(src_h.at[wid], src_v)
    pltpu.sync_copy(dst_h.at[wid], dst_v)
    r0 = s * ROWS_PER_TILE
    pltpu.sync_copy(zeros_h, acc_sh.at[pl.ds(r0, ROWS_PER_TILE)])
    plsc.subcore_barrier()

    def chunk(j, carry):
        pltpu.async_copy(table_h.at[src_v.at[j]], rows_v, sem).wait()
        pltpu.sync_copy(rows_v, acc_sh.at[dst_v.at[j]], add=True)
        return carry

    lax.fori_loop(0, N_CHUNKS, chunk, 0)
    plsc.subcore_barrier()

    pltpu.sync_copy(acc_sh.at[pl.ds(r0, ROWS_PER_TILE)],
                    out_sum.at[c].at[pl.ds(r0, ROWS_PER_TILE)])


_sc_agg = pl.kernel(
    _agg_body,
    out_type=[jax.ShapeDtypeStruct((NC, N_PAD, FEAT), jnp.float32)],
    mesh=_MESH,
    scratch_types=[
        pltpu.VMEM((N_CHUNKS, CHUNK), jnp.int32),
        pltpu.VMEM((N_CHUNKS, CHUNK), jnp.int32),
        pltpu.VMEM((CHUNK, FEAT), jnp.float32),
        pltpu.VMEM_SHARED((N_PAD, FEAT), jnp.float32),
        pltpu.SemaphoreType.DMA,
    ],
)


def _cnt_body(dst_h, zcnt_h, ones_h, out_cnt, dst_v, ones_v, cnt_sh):
    c = lax.axis_index("c")
    s = lax.axis_index("s")
    wid = c * NS + s

    pltpu.sync_copy(dst_h.at[wid], dst_v)
    pltpu.sync_copy(ones_h, ones_v)
    r0 = s * ROWS_PER_TILE
    pltpu.sync_copy(zcnt_h, cnt_sh.at[pl.ds(r0, ROWS_PER_TILE)])
    plsc.subcore_barrier()

    def chunk(j, carry):
        pltpu.sync_copy(ones_v, cnt_sh.at[dst_v.at[j]], add=True)
        return carry

    lax.fori_loop(0, N_CHUNKS, chunk, 0)
    plsc.subcore_barrier()

    pltpu.sync_copy(cnt_sh.at[pl.ds(r0, ROWS_PER_TILE)],
                    out_cnt.at[c].at[pl.ds(r0, ROWS_PER_TILE)])


_sc_counts = pl.kernel(
    _cnt_body,
    out_type=[jax.ShapeDtypeStruct((NC, N_PAD, CNT_W), jnp.float32)],
    mesh=_MESH,
    scratch_types=[
        pltpu.VMEM((N_CHUNKS, CHUNK), jnp.int32),
        pltpu.VMEM((CHUNK, CNT_W), jnp.float32),
        pltpu.VMEM_SHARED((N_PAD, CNT_W), jnp.float32),
    ],
)


BN = 1024


def _tc_layer_body(hs_ref, p0_ref, p1_ref, c0_ref, c1_ref, w_ref, o_ref):
    cnt = c0_ref[:, 0:1] + c1_ref[:, 0:1]
    inv = 1.0 / jnp.maximum(cnt, 1.0)
    neigh = (p0_ref[...] + p1_ref[...]) * inv
    o_ref[...] = (
        jnp.dot(hs_ref[...], w_ref[0], preferred_element_type=jnp.float32)
        + jnp.dot(neigh, w_ref[1], preferred_element_type=jnp.float32)
    )


def _tc_layer(hself, p0, p1, c0, c1, W):
    grid = (N_PAD // BN,)
    return pl.pallas_call(
        _tc_layer_body,
        grid=grid,
        in_specs=[
            pl.BlockSpec((BN, FEAT), lambda i: (i, 0)),
            pl.BlockSpec((BN, FEAT), lambda i: (i, 0)),
            pl.BlockSpec((BN, FEAT), lambda i: (i, 0)),
            pl.BlockSpec((BN, CNT_W), lambda i: (i, 0)),
            pl.BlockSpec((BN, CNT_W), lambda i: (i, 0)),
            pl.BlockSpec((2, FEAT, FEAT), lambda i: (0, 0, 0)),
        ],
        out_specs=pl.BlockSpec((BN, FEAT), lambda i: (i, 0)),
        out_shape=jax.ShapeDtypeStruct((N_PAD, FEAT), jnp.float32),
    )(hself, p0, p1, c0, c1, W)


@jax.jit
def kernel(x, edge_index, W_in, W_out):
    src = edge_index[0].astype(jnp.int32)
    dst = edge_index[1].astype(jnp.int32)
    pad = E_PAD - N_EDGES
    src_p = jnp.concatenate([src, jnp.zeros((pad,), jnp.int32)])
    dst_p = jnp.concatenate([dst, jnp.full((pad,), N_NODES, jnp.int32)])
    src_p = src_p.reshape(NW, N_CHUNKS, CHUNK)
    dst_p = dst_p.reshape(NW, N_CHUNKS, CHUNK)

    x_p = jnp.concatenate(
        [x, jnp.zeros((N_PAD - N_NODES, FEAT), jnp.float32)])
    zeros = jnp.zeros((ROWS_PER_TILE, FEAT), jnp.float32)
    zcnt = jnp.zeros((ROWS_PER_TILE, CNT_W), jnp.float32)
    ones = jnp.ones((CHUNK, CNT_W), jnp.float32)

    (pcnt,) = _sc_counts(dst_p, zcnt, ones)
    (psum0,) = _sc_agg(src_p, dst_p, x_p, zeros)
    h = _tc_layer(x_p, psum0[0], psum0[1], pcnt[0], pcnt[1], W_in)
    (psum1,) = _sc_agg(src_p, dst_p, h, zeros)
    out = _tc_layer(h, psum1[0], psum1[1], pcnt[0], pcnt[1], W_out)
    return out[:N_NODES]
